# Initial kernel scaffold; baseline (speedup 1.0000x reference)
import functools

import jax
import jax.numpy as jnp
from jax import lax
from jax.experimental import pallas as pl
from jax.experimental.pallas import tpu as pltpu

N = 50000
E = 800000
H = 64
L = 3
MAX_ATOM = 100
W = 72

BN = 2500
BE = 2000


def _silu(v):
    return v * jax.nn.sigmoid(v)


def _dotT(a, b):
    return lax.dot_general(a, b, (((1,), (1,)), ((), ())),
                           preferred_element_type=jnp.float32)


def _embed_body(z_ref, x_ref, t_ref, embed_ref, tw1_ref, tb1_ref, tw2_ref,
                tb2_ref, out_ref):
    zcol = z_ref[...]
    iota = lax.broadcasted_iota(jnp.int32, (BN, 128), 1)
    oh = (zcol == iota).astype(jnp.float32)
    h0 = jnp.dot(oh, embed_ref[...], preferred_element_type=jnp.float32)

    t = t_ref[...]
    te = _silu(_dotT(t, tw1_ref[...]) + tb1_ref[...])
    te = _dotT(te, tw2_ref[...]) + tb2_ref[...]
    tvec = jnp.mean(te, axis=0, keepdims=True)

    h0 = h0 + tvec
    out_ref[...] = jnp.concatenate(
        [h0, x_ref[...], jnp.zeros((BN, W - H - 3), jnp.float32)], axis=1)


def _embed_pack(z2, x, t, embed_pad, t_w1, t_b1, t_w2, t_b2):
    grid = (N // BN,)
    return pl.pallas_call(
        _embed_body,
        grid=grid,
        in_specs=[
            pl.BlockSpec((BN, 1), lambda i: (i, 0)),
            pl.BlockSpec((BN, 3), lambda i: (i, 0)),
            pl.BlockSpec((16, 1), lambda i: (0, 0)),
            pl.BlockSpec((128, H), lambda i: (0, 0)),
            pl.BlockSpec((H, 1), lambda i: (0, 0)),
            pl.BlockSpec((1, H), lambda i: (0, 0)),
            pl.BlockSpec((H, H), lambda i: (0, 0)),
            pl.BlockSpec((1, H), lambda i: (0, 0)),
        ],
        out_specs=pl.BlockSpec((BN, W), lambda i: (i, 0)),
        out_shape=jax.ShapeDtypeStruct((N, W), jnp.float32),
    )(z2, x, t, embed_pad, t_w1, t_b1, t_w2, t_b2)


def _edge_body(gs_ref, gd_ref, ew1_ref, eb1_ref, ew2_ref, eb2_ref,
               xw1_ref, xb1_ref, xw2_ref, xb2_ref, out_ref):
    gs = gs_ref[...]
    gd = gd_ref[...]
    hs = gs[:, :H]
    hd = gd[:, :H]
    diff = gd[:, H:H + 3] - gs[:, H:H + 3]
    d2 = jnp.sum(diff * diff, axis=1, keepdims=True)

    ew1 = ew1_ref[...]
    m1 = (_dotT(hd, ew1[:, :H]) + _dotT(hs, ew1[:, H:2 * H])
          + _dotT(d2, ew1[:, 2 * H:2 * H + 1]) + eb1_ref[...])
    m1 = _silu(m1)
    m2 = _silu(_dotT(m1, ew2_ref[...]) + eb2_ref[...])

    p = _silu(_dotT(m2, xw1_ref[...]) + xb1_ref[...])
    cw = _dotT(p, xw2_ref[...]) + xb2_ref[...]

    ndiff = diff / (jnp.sqrt(d2) + 1.0)
    c3 = ndiff * cw
    out_ref[...] = jnp.concatenate(
        [m2, c3, jnp.zeros((BE, W - H - 3), jnp.float32)], axis=1)


def _edge_mlp(gs, gd, ew1, eb1, ew2, eb2, xw1, xb1, xw2, xb2):
    grid = (E // BE,)
    wspec = lambda shape: pl.BlockSpec(shape, lambda i: (0, 0))
    return pl.pallas_call(
        _edge_body,
        grid=grid,
        in_specs=[
            pl.BlockSpec((BE, W), lambda i: (i, 0)),
            pl.BlockSpec((BE, W), lambda i: (i, 0)),
            wspec((H, 2 * H + 1)),
            wspec((1, H)),
            wspec((H, H)),
            wspec((1, H)),
            wspec((H, H)),
            wspec((1, H)),
            wspec((1, H)),
            wspec((1, 1)),
        ],
        out_specs=pl.BlockSpec((BE, W), lambda i: (i, 0)),
        out_shape=jax.ShapeDtypeStruct((E, W), jnp.float32),
    )(gs, gd, ew1, eb1, ew2, eb2, xw1, xb1, xw2, xb2)


def _node_body(t_ref, acc_ref, hw1_ref, hb1_ref, hw2_ref, hb2_ref, out_ref):
    tb = t_ref[...]
    ab = acc_ref[...]
    h = tb[:, :H]
    x = tb[:, H:H + 3]
    agg = ab[:, :H]
    dx = ab[:, H:H + 3]

    xn = x + dx * (1.0 / 16.0)
    hw1 = hw1_ref[...]
    hh = _silu(_dotT(h, hw1[:, :H]) + _dotT(agg, hw1[:, H:]) + hb1_ref[...])
    hn = h + _dotT(hh, hw2_ref[...]) + hb2_ref[...]
    out_ref[...] = jnp.concatenate(
        [hn, xn, jnp.zeros((BN, W - H - 3), jnp.float32)], axis=1)


def _node_update(T, acc, hw1, hb1, hw2, hb2):
    grid = (N // BN,)
    wspec = lambda shape: pl.BlockSpec(shape, lambda i: (0, 0))
    return pl.pallas_call(
        _node_body,
        grid=grid,
        in_specs=[
            pl.BlockSpec((BN, W), lambda i: (i, 0)),
            pl.BlockSpec((BN, W), lambda i: (i, 0)),
            wspec((H, 2 * H)),
            wspec((1, H)),
            wspec((H, H)),
            wspec((1, H)),
        ],
        out_specs=pl.BlockSpec((BN, W), lambda i: (i, 0)),
        out_shape=jax.ShapeDtypeStruct((N, W), jnp.float32),
    )(T, acc, hw1, hb1, hw2, hb2)


def _gather_rows(T, idx):
    return jnp.take(T, idx, axis=0)


def _scatter_add(mc, dst):
    return jax.ops.segment_sum(mc, dst, num_segments=N)


def kernel(x, z, t, edge_index, embed, t_w1, t_b1, t_w2, t_b2,
           e_w1, e_b1, e_w2, e_b2, x_w1, x_b1, x_w2, x_b2,
           h_w1, h_b1, h_w2, h_b2):
    z2 = z.astype(jnp.int32).reshape(N, 1)
    src = edge_index[0].astype(jnp.int32)
    dst = edge_index[1].astype(jnp.int32)
    embed_pad = jnp.pad(embed, ((0, 128 - MAX_ATOM), (0, 0)))

    T = _embed_pack(z2, x, t, embed_pad, t_w1, t_b1.reshape(1, H),
                    t_w2, t_b2.reshape(1, H))

    for l in range(L):
        gs = _gather_rows(T, src)
        gd = _gather_rows(T, dst)
        mc = _edge_mlp(gs, gd, e_w1[l], e_b1[l].reshape(1, H),
                       e_w2[l], e_b2[l].reshape(1, H),
                       x_w1[l], x_b1[l].reshape(1, H),
                       x_w2[l], x_b2[l].reshape(1, 1))
        acc = _scatter_add(mc, dst)
        T = _node_update(T, acc, h_w1[l], h_b1[l].reshape(1, H),
                         T if False else h_w2[l], h_b2[l].reshape(1, H))

    return T[:, H:H + 3]


# speedup vs baseline: 1.3094x; 1.3094x over previous
import functools

import jax
import jax.numpy as jnp
from jax import lax
from jax.experimental import pallas as pl
from jax.experimental.pallas import tpu as pltpu

N = 50000
E = 800000
H = 64
L = 3
MAX_ATOM = 100
W = 72

BN = 2000
BE = 2000


def _silu(v):
    return v * jax.nn.sigmoid(v)


def _dotT(a, b):
    return lax.dot_general(a, b, (((1,), (1,)), ((), ())),
                           preferred_element_type=jnp.float32)


def _embed_body(z_ref, x_ref, t_ref, embed_ref, tw1_ref, tb1_ref, tw2_ref,
                tb2_ref, out_ref):
    zcol = z_ref[...]
    iota = lax.broadcasted_iota(jnp.int32, (BN, 128), 1)
    oh = (zcol == iota).astype(jnp.float32)
    h0 = jnp.dot(oh, embed_ref[...], preferred_element_type=jnp.float32)

    t = t_ref[...]
    te = _silu(_dotT(t, tw1_ref[...]) + tb1_ref[...])
    te = _dotT(te, tw2_ref[...]) + tb2_ref[...]
    tvec = jnp.mean(te, axis=0, keepdims=True)

    h0 = h0 + tvec
    out_ref[...] = jnp.concatenate(
        [h0, x_ref[...], jnp.zeros((BN, W - H - 3), jnp.float32)], axis=1)


def _embed_pack(z2, x, t, embed_pad, t_w1, t_b1, t_w2, t_b2):
    grid = (N // BN,)
    return pl.pallas_call(
        _embed_body,
        grid=grid,
        in_specs=[
            pl.BlockSpec((BN, 1), lambda i: (i, 0)),
            pl.BlockSpec((BN, 3), lambda i: (i, 0)),
            pl.BlockSpec((16, 1), lambda i: (0, 0)),
            pl.BlockSpec((128, H), lambda i: (0, 0)),
            pl.BlockSpec((H, 1), lambda i: (0, 0)),
            pl.BlockSpec((1, H), lambda i: (0, 0)),
            pl.BlockSpec((H, H), lambda i: (0, 0)),
            pl.BlockSpec((1, H), lambda i: (0, 0)),
        ],
        out_specs=pl.BlockSpec((BN, W), lambda i: (i, 0)),
        out_shape=jax.ShapeDtypeStruct((N, W), jnp.float32),
    )(z2, x, t, embed_pad, t_w1, t_b1, t_w2, t_b2)


def _edge_body(gs_ref, gd_ref, ew1_ref, eb1_ref, ew2_ref, eb2_ref,
               xw1_ref, xb1_ref, xw2_ref, xb2_ref, out_ref):
    gs = gs_ref[...]
    gd = gd_ref[...]
    hs = gs[:, :H]
    hd = gd[:, :H]
    diff = gd[:, H:H + 3] - gs[:, H:H + 3]
    d2 = jnp.sum(diff * diff, axis=1, keepdims=True)

    ew1 = ew1_ref[...]
    m1 = (_dotT(hd, ew1[:, :H]) + _dotT(hs, ew1[:, H:2 * H])
          + _dotT(d2, ew1[:, 2 * H:2 * H + 1]) + eb1_ref[...])
    m1 = _silu(m1)
    m2 = _silu(_dotT(m1, ew2_ref[...]) + eb2_ref[...])

    p = _silu(_dotT(m2, xw1_ref[...]) + xb1_ref[...])
    cw = (jnp.sum(p * xw2_ref[...], axis=1, keepdims=True)
          + xb2_ref[0, 0])

    ndiff = diff / (jnp.sqrt(d2) + 1.0)
    c3 = ndiff * cw
    out_ref[...] = jnp.concatenate(
        [m2, c3, jnp.zeros((BE, W - H - 3), jnp.float32)], axis=1)


def _edge_mlp(gs, gd, ew1, eb1, ew2, eb2, xw1, xb1, xw2, xb2):
    grid = (E // BE,)
    wspec = lambda shape: pl.BlockSpec(shape, lambda i: (0, 0))
    return pl.pallas_call(
        _edge_body,
        grid=grid,
        in_specs=[
            pl.BlockSpec((BE, W), lambda i: (i, 0)),
            pl.BlockSpec((BE, W), lambda i: (i, 0)),
            wspec((H, 2 * H + 1)),
            wspec((1, H)),
            wspec((H, H)),
            wspec((1, H)),
            wspec((H, H)),
            wspec((1, H)),
            wspec((1, H)),
            wspec((1, 1)),
        ],
        out_specs=pl.BlockSpec((BE, W), lambda i: (i, 0)),
        out_shape=jax.ShapeDtypeStruct((E, W), jnp.float32),
    )(gs, gd, ew1, eb1, ew2, eb2, xw1, xb1, xw2, xb2)


def _node_body(t_ref, acc_ref, hw1_ref, hb1_ref, hw2_ref, hb2_ref, out_ref):
    tb = t_ref[...]
    ab = acc_ref[...]
    h = tb[:, :H]
    x = tb[:, H:H + 3]
    agg = ab[:, :H]
    dx = ab[:, H:H + 3]

    xn = x + dx * (1.0 / 16.0)
    hw1 = hw1_ref[...]
    hh = _silu(_dotT(h, hw1[:, :H]) + _dotT(agg, hw1[:, H:]) + hb1_ref[...])
    hn = h + _dotT(hh, hw2_ref[...]) + hb2_ref[...]
    out_ref[...] = jnp.concatenate(
        [hn, xn, jnp.zeros((BN, W - H - 3), jnp.float32)], axis=1)


def _node_update(T, acc, hw1, hb1, hw2, hb2):
    grid = (N // BN,)
    wspec = lambda shape: pl.BlockSpec(shape, lambda i: (0, 0))
    return pl.pallas_call(
        _node_body,
        grid=grid,
        in_specs=[
            pl.BlockSpec((BN, W), lambda i: (i, 0)),
            pl.BlockSpec((BN, W), lambda i: (i, 0)),
            wspec((H, 2 * H)),
            wspec((1, H)),
            wspec((H, H)),
            wspec((1, H)),
        ],
        out_specs=pl.BlockSpec((BN, W), lambda i: (i, 0)),
        out_shape=jax.ShapeDtypeStruct((N, W), jnp.float32),
    )(T, acc, hw1, hb1, hw2, hb2)


def _gather_rows(T, idx):
    return jnp.take(T, idx, axis=0)


def _scatter_add(mc, dst):
    return jax.ops.segment_sum(mc, dst, num_segments=N)


def kernel(x, z, t, edge_index, embed, t_w1, t_b1, t_w2, t_b2,
           e_w1, e_b1, e_w2, e_b2, x_w1, x_b1, x_w2, x_b2,
           h_w1, h_b1, h_w2, h_b2):
    z2 = z.astype(jnp.int32).reshape(N, 1)
    src = edge_index[0].astype(jnp.int32)
    dst = edge_index[1].astype(jnp.int32)
    embed_pad = jnp.pad(embed, ((0, 128 - MAX_ATOM), (0, 0)))

    T = _embed_pack(z2, x, t, embed_pad, t_w1, t_b1.reshape(1, H),
                    t_w2, t_b2.reshape(1, H))

    for l in range(L):
        gs = _gather_rows(T, src)
        gd = _gather_rows(T, dst)
        mc = _edge_mlp(gs, gd, e_w1[l], e_b1[l].reshape(1, H),
                       e_w2[l], e_b2[l].reshape(1, H),
                       x_w1[l], x_b1[l].reshape(1, H),
                       x_w2[l], x_b2[l].reshape(1, 1))
        acc = _scatter_add(mc, dst)
        T = _node_update(T, acc, h_w1[l], h_b1[l].reshape(1, H),
                         h_w2[l], h_b2[l].reshape(1, H))

    return T[:, H:H + 3]


# speedup vs baseline: 2.3072x; 1.7620x over previous
import functools

import jax
import jax.numpy as jnp
from jax import lax
from jax.experimental import pallas as pl
from jax.experimental.pallas import tpu as pltpu
from jax.experimental.pallas import tpu_sc as plsc

N = 50000
E = 800000
H = 64
L = 3
MAX_ATOM = 100
W = 72
WT = 128

BN = 2000
BE = 2000


def _silu(v):
    return v * jax.nn.sigmoid(v)


def _dotT(a, b):
    return lax.dot_general(a, b, (((1,), (1,)), ((), ())),
                           preferred_element_type=jnp.float32)


def _embed_body(z_ref, x_ref, t_ref, embed_ref, tw1_ref, tb1_ref, tw2_ref,
                tb2_ref, out_ref):
    zcol = z_ref[...]
    iota = lax.broadcasted_iota(jnp.int32, (BN, 128), 1)
    oh = (zcol == iota).astype(jnp.float32)
    h0 = jnp.dot(oh, embed_ref[...], preferred_element_type=jnp.float32)

    t = t_ref[...]
    te = _silu(_dotT(t, tw1_ref[...]) + tb1_ref[...])
    te = _dotT(te, tw2_ref[...]) + tb2_ref[...]
    tvec = jnp.mean(te, axis=0, keepdims=True)

    h0 = h0 + tvec
    out_ref[...] = jnp.concatenate(
        [h0, x_ref[...], jnp.zeros((BN, WT - H - 3), jnp.float32)], axis=1)


def _embed_pack(z2, x, t, embed_pad, t_w1, t_b1, t_w2, t_b2):
    grid = (N // BN,)
    return pl.pallas_call(
        _embed_body,
        grid=grid,
        in_specs=[
            pl.BlockSpec((BN, 1), lambda i: (i, 0)),
            pl.BlockSpec((BN, 3), lambda i: (i, 0)),
            pl.BlockSpec((16, 1), lambda i: (0, 0)),
            pl.BlockSpec((128, H), lambda i: (0, 0)),
            pl.BlockSpec((H, 1), lambda i: (0, 0)),
            pl.BlockSpec((1, H), lambda i: (0, 0)),
            pl.BlockSpec((H, H), lambda i: (0, 0)),
            pl.BlockSpec((1, H), lambda i: (0, 0)),
        ],
        out_specs=pl.BlockSpec((BN, WT), lambda i: (i, 0)),
        out_shape=jax.ShapeDtypeStruct((N, WT), jnp.float32),
    )(z2, x, t, embed_pad, t_w1, t_b1, t_w2, t_b2)


def _edge_body(gs_ref, gd_ref, ew1_ref, eb1_ref, ew2_ref, eb2_ref,
               xw1_ref, xb1_ref, xw2_ref, xb2_ref, out_ref):
    gs = gs_ref[...]
    gd = gd_ref[...]
    hs = gs[:, :H]
    hd = gd[:, :H]
    diff = gd[:, H:H + 3] - gs[:, H:H + 3]
    d2 = jnp.sum(diff * diff, axis=1, keepdims=True)

    ew1 = ew1_ref[...]
    m1 = (_dotT(hd, ew1[:, :H]) + _dotT(hs, ew1[:, H:2 * H])
          + _dotT(d2, ew1[:, 2 * H:2 * H + 1]) + eb1_ref[...])
    m1 = _silu(m1)
    m2 = _silu(_dotT(m1, ew2_ref[...]) + eb2_ref[...])

    p = _silu(_dotT(m2, xw1_ref[...]) + xb1_ref[...])
    cw = (jnp.sum(p * xw2_ref[...], axis=1, keepdims=True)
          + xb2_ref[0, 0])

    ndiff = diff / (jnp.sqrt(d2) + 1.0)
    c3 = ndiff * cw
    out_ref[...] = jnp.concatenate(
        [m2, c3, jnp.zeros((BE, W - H - 3), jnp.float32)], axis=1)


def _edge_mlp(gs, gd, ew1, eb1, ew2, eb2, xw1, xb1, xw2, xb2):
    grid = (E // BE,)
    wspec = lambda shape: pl.BlockSpec(shape, lambda i: (0, 0))
    return pl.pallas_call(
        _edge_body,
        grid=grid,
        in_specs=[
            pl.BlockSpec((BE, WT), lambda i: (i, 0)),
            pl.BlockSpec((BE, WT), lambda i: (i, 0)),
            wspec((H, 2 * H + 1)),
            wspec((1, H)),
            wspec((H, H)),
            wspec((1, H)),
            wspec((H, H)),
            wspec((1, H)),
            wspec((1, H)),
            wspec((1, 1)),
        ],
        out_specs=pl.BlockSpec((BE, W), lambda i: (i, 0)),
        out_shape=jax.ShapeDtypeStruct((E, W), jnp.float32),
    )(gs, gd, ew1, eb1, ew2, eb2, xw1, xb1, xw2, xb2)


def _node_body(t_ref, acc_ref, hw1_ref, hb1_ref, hw2_ref, hb2_ref, out_ref):
    tb = t_ref[...]
    ab = acc_ref[...]
    h = tb[:, :H]
    x = tb[:, H:H + 3]
    agg = ab[:, :H]
    dx = ab[:, H:H + 3]

    xn = x + dx * (1.0 / 16.0)
    hw1 = hw1_ref[...]
    hh = _silu(_dotT(h, hw1[:, :H]) + _dotT(agg, hw1[:, H:]) + hb1_ref[...])
    hn = h + _dotT(hh, hw2_ref[...]) + hb2_ref[...]
    out_ref[...] = jnp.concatenate(
        [hn, xn, jnp.zeros((BN, WT - H - 3), jnp.float32)], axis=1)


def _node_update(T, acc, hw1, hb1, hw2, hb2):
    grid = (N // BN,)
    wspec = lambda shape: pl.BlockSpec(shape, lambda i: (0, 0))
    return pl.pallas_call(
        _node_body,
        grid=grid,
        in_specs=[
            pl.BlockSpec((BN, WT), lambda i: (i, 0)),
            pl.BlockSpec((BN, W), lambda i: (i, 0)),
            wspec((H, 2 * H)),
            wspec((1, H)),
            wspec((H, H)),
            wspec((1, H)),
        ],
        out_specs=pl.BlockSpec((BN, WT), lambda i: (i, 0)),
        out_shape=jax.ShapeDtypeStruct((N, WT), jnp.float32),
    )(T, acc, hw1, hb1, hw2, hb2)


NC = 2
NS = 16
NW = NC * NS
EPW = E // NW
GCH = 128
GFULL = EPW // GCH
GTAIL = EPW - GFULL * GCH

_MESH = plsc.VectorSubcoreMesh(core_axis_name="c", subcore_axis_name="s")


def _gather_body(t_hbm, src_hbm, dst_hbm, gs_hbm, gd_hbm,
                 idxs_v, idxd_v, rows_v, rowt_v, sem):
    c = lax.axis_index("c")
    s = lax.axis_index("s")
    wid = s * NC + c
    base = wid * EPW
    pltpu.sync_copy(src_hbm.at[pl.ds(base, EPW)], idxs_v)
    pltpu.sync_copy(dst_hbm.at[pl.ds(base, EPW)], idxd_v)

    def chunk(i, _):
        off = i * GCH
        pltpu.async_copy(t_hbm.at[idxs_v.at[pl.ds(off, GCH)]], rows_v,
                         sem).wait()
        pltpu.sync_copy(rows_v, gs_hbm.at[pl.ds(base + off, GCH)])
        pltpu.async_copy(t_hbm.at[idxd_v.at[pl.ds(off, GCH)]], rows_v,
                         sem).wait()
        pltpu.sync_copy(rows_v, gd_hbm.at[pl.ds(base + off, GCH)])
        return 0

    lax.fori_loop(0, GFULL, chunk, 0)
    toff = GFULL * GCH
    pltpu.async_copy(t_hbm.at[idxs_v.at[pl.ds(toff, GTAIL)]], rowt_v,
                     sem).wait()
    pltpu.sync_copy(rowt_v, gs_hbm.at[pl.ds(base + toff, GTAIL)])
    pltpu.async_copy(t_hbm.at[idxd_v.at[pl.ds(toff, GTAIL)]], rowt_v,
                     sem).wait()
    pltpu.sync_copy(rowt_v, gd_hbm.at[pl.ds(base + toff, GTAIL)])


def _gather_rows2(T, src, dst):
    return pl.kernel(
        _gather_body,
        out_type=[jax.ShapeDtypeStruct((E, WT), jnp.float32),
                  jax.ShapeDtypeStruct((E, WT), jnp.float32)],
        mesh=_MESH,
        scratch_types=[
            pltpu.VMEM((EPW,), jnp.int32),
            pltpu.VMEM((EPW,), jnp.int32),
            pltpu.VMEM((GCH, WT), jnp.float32),
            pltpu.VMEM((GTAIL, WT), jnp.float32),
            pltpu.SemaphoreType.DMA,
        ],
    )(T, src, dst)


NPC = N // NC
ZROWS = 1563
SPROWS = NS * ZROWS
EPS = E // NS
SCH = 80
SNCH = EPS // SCH
CPT = 1560
CREM = NPC - NS * CPT


def _scatter_body(mc_hbm, dst_hbm, zeros_hbm, acc_hbm,
                  idx_v, adj_v, rows_v, acc_sp):
    c = lax.axis_index("c")
    s = lax.axis_index("s")
    nodebase = c * NPC

    pltpu.sync_copy(zeros_hbm, acc_sp.at[pl.ds(s * ZROWS, ZROWS)])
    plsc.subcore_barrier()

    ebase = s * EPS

    def chunk(ci, _):
        off = ebase + ci * SCH
        pltpu.sync_copy(dst_hbm.at[pl.ds(off, SCH)], idx_v)
        pltpu.sync_copy(mc_hbm.at[pl.ds(off, SCH)], rows_v)

        def it(j, _):
            v = idx_v[pl.ds(j * 16, 16)]
            lv = v - nodebase
            ok = (lv >= 0) & (lv < NPC)
            adj_v[pl.ds(j * 16, 16)] = jnp.where(ok, lv, NPC)
            return 0
        lax.fori_loop(0, SCH // 16, it, 0)
        pltpu.sync_copy(rows_v, acc_sp.at[adj_v], add=True)
        return 0

    lax.fori_loop(0, SNCH, chunk, 0)

    plsc.subcore_barrier()
    pltpu.sync_copy(acc_sp.at[pl.ds(s * CPT, CPT)],
                    acc_hbm.at[pl.ds(nodebase + s * CPT, CPT)])

    @pl.when(s == NS - 1)
    def _():
        pltpu.sync_copy(acc_sp.at[pl.ds(NS * CPT, CREM)],
                        acc_hbm.at[pl.ds(nodebase + NS * CPT, CREM)])


def _scatter_add2(mc, dst, zeros_blk):
    return pl.kernel(
        _scatter_body,
        out_type=jax.ShapeDtypeStruct((N, W), jnp.float32),
        mesh=_MESH,
        scratch_types=[
            pltpu.VMEM((SCH,), jnp.int32),
            pltpu.VMEM((SCH,), jnp.int32),
            pltpu.VMEM((SCH, W), jnp.float32),
            pltpu.VMEM_SHARED((SPROWS, W), jnp.float32),
        ],
    )(mc, dst, zeros_blk)


def kernel(x, z, t, edge_index, embed, t_w1, t_b1, t_w2, t_b2,
           e_w1, e_b1, e_w2, e_b2, x_w1, x_b1, x_w2, x_b2,
           h_w1, h_b1, h_w2, h_b2):
    z2 = z.astype(jnp.int32).reshape(N, 1)
    src = edge_index[0].astype(jnp.int32)
    dst = edge_index[1].astype(jnp.int32)
    embed_pad = jnp.pad(embed, ((0, 128 - MAX_ATOM), (0, 0)))
    zeros_blk = jnp.zeros((ZROWS, W), jnp.float32)

    T = _embed_pack(z2, x, t, embed_pad, t_w1, t_b1.reshape(1, H),
                    t_w2, t_b2.reshape(1, H))

    for l in range(L):
        gs, gd = _gather_rows2(T, src, dst)
        mc = _edge_mlp(gs, gd, e_w1[l], e_b1[l].reshape(1, H),
                       e_w2[l], e_b2[l].reshape(1, H),
                       x_w1[l], x_b1[l].reshape(1, H),
                       x_w2[l], x_b2[l].reshape(1, 1))
        acc = jax.ops.segment_sum(mc, dst, num_segments=N)
        T = _node_update(T, acc, h_w1[l], h_b1[l].reshape(1, H),
                         h_w2[l], h_b2[l].reshape(1, H))

    return T[:, H:H + 3]
